# baseline (speedup 1.0000x reference)
import functools

import jax
import jax.numpy as jnp
from jax import lax
from jax.experimental import pallas as pl
from jax.experimental.pallas import tpu as pltpu
from jax.experimental.pallas import tpu_sc as plsc

N_NODES = 10000
N_EDGES = 320000
NC, NS = 2, 16
NW = NC * NS
K = 128
CH_W = 80
NB = 8
E_PAD = NW * CH_W * K
W = 128
SINK = N_NODES
N_SINK = 240
N_ACC = N_NODES + N_SINK

ROWS_Z = N_ACC // NS
ROWS_LAST_D = N_NODES - (NS - 1) * ROWS_Z


def _zero_or_dump(sid, zeros_hbm, acc, out_ref=None, cid=None):
  off = sid * ROWS_Z

  if out_ref is None:
    pltpu.sync_copy(zeros_hbm, acc.at[pl.ds(off, ROWS_Z)])
    return

  @pl.when(sid < NS - 1)
  def _():
    pltpu.sync_copy(acc.at[pl.ds(off, ROWS_Z)],
                    out_ref.at[cid, pl.ds(off, ROWS_Z)])

  @pl.when(sid == NS - 1)
  def _():
    pltpu.sync_copy(acc.at[pl.ds(off, ROWS_LAST_D)],
                    out_ref.at[cid, pl.ds(off, ROWS_LAST_D)])


def _make_feat_scatter():
  mesh = plsc.VectorSubcoreMesh(core_axis_name="c", subcore_axis_name="s")

  @functools.partial(
      pl.kernel,
      out_type=jax.ShapeDtypeStruct((NC, N_NODES, W), jnp.float32),
      mesh=mesh,
      scratch_types=(
          [pltpu.VMEM((K,), jnp.int32) for _ in range(2 * NB)]
          + [pltpu.VMEM((K, W), jnp.float32), pltpu.VMEM((K, W), jnp.float32)]
          + [pltpu.VMEM_SHARED((N_ACC, W), jnp.float32)]
          + [pltpu.SemaphoreType.DMA for _ in range(NB + 2)]
      ),
  )
  def k(y_hbm, src_hbm, dst_hbm, zeros_hbm, out_hbm, *refs):
    idx = refs[:2 * NB]
    rows = refs[2 * NB:2 * NB + 2]
    acc = refs[2 * NB + 2]
    sem_i = refs[2 * NB + 3:2 * NB + 3 + NB]
    sem_g = refs[2 * NB + 3 + NB:]
    cid = lax.axis_index("c")
    sid = lax.axis_index("s")
    wid = cid * NS + sid

    _zero_or_dump(sid, zeros_hbm, acc)

    def idx_slices(j):
      base = (wid * CH_W + j) * K
      return src_hbm.at[pl.ds(base, K)], dst_hbm.at[pl.ds(base, K)]

    def fire_idx(j, b):
      s, d = idx_slices(j)
      pltpu.async_copy(s, idx[2 * b], sem_i[b])
      pltpu.async_copy(d, idx[2 * b + 1], sem_i[b])

    def wait_idx(j, b):
      s, d = idx_slices(j)
      pltpu.make_async_copy(s, idx[2 * b], sem_i[b]).wait()
      pltpu.make_async_copy(d, idx[2 * b + 1], sem_i[b]).wait()

    def gather(b, r, sem):
      pltpu.async_copy(y_hbm.at[idx[2 * b]], rows[r], sem)

    def wait_gather(b, r, sem):
      pltpu.make_async_copy(y_hbm.at[idx[2 * b]], rows[r], sem).wait()

    def scatter(b, r):
      pltpu.sync_copy(rows[r], acc.at[idx[2 * b + 1]], add=True)

    for b in range(NB):
      fire_idx(b, b)
    plsc.subcore_barrier()
    wait_idx(0, 0)
    gather(0, 0, sem_g[0])

    def body(q, carry):
      j = q * NB
      for b in range(NB):
        r = b % 2
        wait_gather(b, r, sem_g[r])
        if b < NB - 1:
          wait_idx(j + b + 1, b + 1)
          gather(b + 1, 1 - r, sem_g[1 - r])
        else:
          @pl.when(j + NB < CH_W)
          def _():
            wait_idx(j + NB, 0)
            gather(0, 1 - r, sem_g[1 - r])

        scatter(b, r)

        @pl.when(j + NB + b < CH_W)
        def _():
          fire_idx(j + NB + b, b)
      return carry

    lax.fori_loop(0, CH_W // NB, body, 0)

    plsc.subcore_barrier()
    _zero_or_dump(sid, None, acc, out_ref=out_hbm, cid=cid)

  return k


def _bn(h, gamma, beta, eps=1e-5):
  m = jnp.mean(h, axis=0, keepdims=True)
  v = jnp.mean((h - m) ** 2, axis=0, keepdims=True)
  return gamma * (h - m) * jax.lax.rsqrt(v + eps) + beta


def _k0_body(x_ref, wl_ref, y_ref):
  x = x_ref[...]
  n = x.shape[0]
  y = jnp.dot(x, wl_ref[...], preferred_element_type=jnp.float32)
  pad1 = jnp.ones((n, 1), jnp.float32)
  pad0 = jnp.zeros((n, 63), jnp.float32)
  y_ref[...] = jnp.concatenate([y, pad1, pad0], axis=1)


def _z_body(h_ref, wr_ref, b_ref, z_ref):
  z_ref[...] = (jnp.dot(h_ref[...], wr_ref[...],
                        preferred_element_type=jnp.float32) + b_ref[...])


def _m0_body(s_ref, z_ref, g_ref, be_ref, h_ref, inv_ref):
  s = s_ref[0] + s_ref[1]
  cnt = s[:, 64:65]
  inv = 1.0 / jnp.maximum(cnt, 1.0)
  inv_ref[...] = jnp.broadcast_to(inv, inv_ref.shape)
  h = s * inv + z_ref[...]
  h_ref[...] = jnp.maximum(_bn(h, g_ref[...], be_ref[...]), 0.0)


def _m_body(s_ref, inv_ref, wl_ref, z_ref, g_ref, be_ref, o_ref):
  inv = inv_ref[:, 0:1]
  mean = (s_ref[0] + s_ref[1]) * inv
  h = jnp.dot(mean, wl_ref[...], preferred_element_type=jnp.float32) + z_ref[...]
  o_ref[...] = jnp.maximum(_bn(h, g_ref[...], be_ref[...]), 0.0)


def _mw_body(sa_ref, sb_ref, inv_ref, wla_ref, wlb_ref, z_ref, g_ref, be_ref,
             o_ref):
  inv = inv_ref[:, 0:1]
  mean_a = (sa_ref[0] + sa_ref[1]) * inv
  mean_b = (sb_ref[0] + sb_ref[1]) * inv
  h = (jnp.dot(mean_a, wla_ref[...], preferred_element_type=jnp.float32)
       + jnp.dot(mean_b, wlb_ref[...], preferred_element_type=jnp.float32)
       + z_ref[...])
  o_ref[...] = jnp.maximum(_bn(h, g_ref[...], be_ref[...]), 0.0)


def _ml_body(sa_ref, sb_ref, inv_ref, wla_ref, wlb_ref, z_ref, o_ref):
  inv = inv_ref[:, 0:1]
  mean_a = (sa_ref[0] + sa_ref[1]) * inv
  mean_b = (sb_ref[0] + sb_ref[1]) * inv
  o_ref[...] = (jnp.dot(mean_a, wla_ref[...], preferred_element_type=jnp.float32)
                + jnp.dot(mean_b, wlb_ref[...], preferred_element_type=jnp.float32)
                + z_ref[...])


def kernel(x, edge_index, params):
  n = x.shape[0]
  f32 = jnp.float32
  pad = E_PAD - N_EDGES
  pad_ar = jnp.arange(pad, dtype=jnp.int32)
  src = jnp.concatenate([edge_index[0], pad_ar % N_NODES])
  dst = jnp.concatenate([edge_index[1], SINK + pad_ar % N_SINK])

  zeros128 = jnp.zeros((ROWS_Z, W), f32)
  scat = _make_feat_scatter()
  sds = jax.ShapeDtypeStruct

  def pad_rows(w, rows=W):
    return jnp.pad(w, ((0, rows - w.shape[0]), (0, 0)))

  def pad_cols(v, cols, value=0.0):
    return jnp.pad(v.reshape(1, -1), ((0, 0), (0, cols - v.shape[0])),
                   constant_values=value)

  def zk(h, wr, b, dout):
    return pl.pallas_call(_z_body, out_shape=sds((n, dout), f32))(
        h, wr, b.reshape(1, dout))

  y0 = pl.pallas_call(_k0_body, out_shape=sds((n, W), f32))(x, params["Wl0"])
  s0 = scat(y0, src, dst, zeros128)
  wr0p = jnp.pad(params["Wr0"], ((0, 0), (0, W - 64)))
  z0 = zk(x, wr0p, pad_cols(params["b0"], W).reshape(-1), W)
  h1, inv = pl.pallas_call(
      _m0_body,
      out_shape=(sds((n, W), f32), sds((n, 16), f32)),
  )(s0, z0, pad_cols(params["gamma0"], W, value=1.0), pad_cols(params["beta0"], W))

  s1 = scat(h1, src, dst, zeros128)
  z1 = zk(h1, pad_rows(params["Wr1"]), params["b1"], 128)
  h2 = pl.pallas_call(
      _m_body,
      out_shape=sds((n, 128), f32),
  )(s1, inv, pad_rows(params["Wl1"]), z1,
    params["gamma1"].reshape(1, 128), params["beta1"].reshape(1, 128))

  s2 = scat(h2, src, dst, zeros128)
  z2 = zk(h2, params["Wr2"], params["b2"], 256)
  h3 = pl.pallas_call(
      _m_body,
      out_shape=sds((n, 256), f32),
  )(s2, inv, params["Wl2"], z2,
    params["gamma2"].reshape(1, 256), params["beta2"].reshape(1, 256))

  s3a = scat(h3[:, :128], src, dst, zeros128)
  s3b = scat(h3[:, 128:], src, dst, zeros128)
  z3 = zk(h3, params["Wr3"], params["b3"], 256)
  h4 = pl.pallas_call(
      _mw_body,
      out_shape=sds((n, 256), f32),
  )(s3a, s3b, inv, params["Wl3"][:128], params["Wl3"][128:], z3,
    params["gamma3"].reshape(1, 256), params["beta3"].reshape(1, 256))

  s4a = scat(h4[:, :128], src, dst, zeros128)
  s4b = scat(h4[:, 128:], src, dst, zeros128)
  mb = 2000
  z4 = pl.pallas_call(
      _z_body,
      grid=(n // mb,),
      in_specs=[
          pl.BlockSpec((mb, 256), lambda i: (i, 0)),
          pl.BlockSpec((256, 576), lambda i: (0, 0)),
          pl.BlockSpec((1, 576), lambda i: (0, 0)),
      ],
      out_specs=pl.BlockSpec((mb, 576), lambda i: (i, 0)),
      out_shape=sds((n, 576), f32),
  )(h4, params["Wr4"], params["b4"].reshape(1, 576))
  out = pl.pallas_call(
      _ml_body,
      grid=(n // mb,),
      in_specs=[
          pl.BlockSpec((2, mb, 128), lambda i: (0, i, 0)),
          pl.BlockSpec((2, mb, 128), lambda i: (0, i, 0)),
          pl.BlockSpec((mb, 16), lambda i: (i, 0)),
          pl.BlockSpec((128, 576), lambda i: (0, 0)),
          pl.BlockSpec((128, 576), lambda i: (0, 0)),
          pl.BlockSpec((mb, 576), lambda i: (i, 0)),
      ],
      out_specs=pl.BlockSpec((mb, 576), lambda i: (i, 0)),
      out_shape=sds((n, 576), f32),
  )(s4a, s4b, inv, params["Wl4"][:128], params["Wl4"][128:], z4)
  return out


# speedup vs baseline: 1.0410x; 1.0410x over previous
import functools

import jax
import jax.numpy as jnp
from jax import lax
from jax.experimental import pallas as pl
from jax.experimental.pallas import tpu as pltpu
from jax.experimental.pallas import tpu_sc as plsc

N_NODES = 10000
N_EDGES = 320000
NC, NS = 2, 16
NW = NC * NS
K = 128
CH_W = 80
NB = 8
E_PAD = NW * CH_W * K
W = 128
SINK = N_NODES
N_SINK = 240
N_ACC = N_NODES + N_SINK

ROWS_Z = N_ACC // NS
ROWS_LAST_D = N_NODES - (NS - 1) * ROWS_Z


def _zero_or_dump(sid, zeros_hbm, acc, out_ref=None, cid=None):
  off = sid * ROWS_Z

  if out_ref is None:
    pltpu.sync_copy(zeros_hbm, acc.at[pl.ds(off, ROWS_Z)])
    return

  @pl.when(sid < NS - 1)
  def _():
    pltpu.sync_copy(acc.at[pl.ds(off, ROWS_Z)],
                    out_ref.at[cid, pl.ds(off, ROWS_Z)])

  @pl.when(sid == NS - 1)
  def _():
    pltpu.sync_copy(acc.at[pl.ds(off, ROWS_LAST_D)],
                    out_ref.at[cid, pl.ds(off, ROWS_LAST_D)])


def _make_feat_scatter(W=W, tc_tiling=True):
  mesh = plsc.VectorSubcoreMesh(core_axis_name="c", subcore_axis_name="s")
  cp = None if tc_tiling else pltpu.CompilerParams(use_tc_tiling_on_sc=False)

  @functools.partial(
      pl.kernel,
      out_type=jax.ShapeDtypeStruct((NC, N_NODES, W), jnp.float32),
      mesh=mesh,
      compiler_params=cp,
      scratch_types=(
          [pltpu.VMEM((K,), jnp.int32) for _ in range(2 * NB)]
          + [pltpu.VMEM((K, W), jnp.float32), pltpu.VMEM((K, W), jnp.float32)]
          + [pltpu.VMEM_SHARED((N_ACC, W), jnp.float32)]
          + [pltpu.SemaphoreType.DMA for _ in range(NB + 2)]
      ),
  )
  def k(y_hbm, src_hbm, dst_hbm, zeros_hbm, out_hbm, *refs):
    idx = refs[:2 * NB]
    rows = refs[2 * NB:2 * NB + 2]
    acc = refs[2 * NB + 2]
    sem_i = refs[2 * NB + 3:2 * NB + 3 + NB]
    sem_g = refs[2 * NB + 3 + NB:]
    cid = lax.axis_index("c")
    sid = lax.axis_index("s")
    wid = cid * NS + sid

    _zero_or_dump(sid, zeros_hbm, acc)

    def idx_slices(j):
      base = (wid * CH_W + j) * K
      return src_hbm.at[pl.ds(base, K)], dst_hbm.at[pl.ds(base, K)]

    def fire_idx(j, b):
      s, d = idx_slices(j)
      pltpu.async_copy(s, idx[2 * b], sem_i[b])
      pltpu.async_copy(d, idx[2 * b + 1], sem_i[b])

    def wait_idx(j, b):
      s, d = idx_slices(j)
      pltpu.make_async_copy(s, idx[2 * b], sem_i[b]).wait()
      pltpu.make_async_copy(d, idx[2 * b + 1], sem_i[b]).wait()

    def gather(b, r, sem):
      pltpu.async_copy(y_hbm.at[idx[2 * b]], rows[r], sem)

    def wait_gather(b, r, sem):
      pltpu.make_async_copy(y_hbm.at[idx[2 * b]], rows[r], sem).wait()

    def scatter(b, r):
      pltpu.sync_copy(rows[r], acc.at[idx[2 * b + 1]], add=True)

    for b in range(NB):
      fire_idx(b, b)
    plsc.subcore_barrier()
    wait_idx(0, 0)
    gather(0, 0, sem_g[0])

    def body(q, carry):
      j = q * NB
      for b in range(NB):
        r = b % 2
        wait_gather(b, r, sem_g[r])
        if b < NB - 1:
          wait_idx(j + b + 1, b + 1)
          gather(b + 1, 1 - r, sem_g[1 - r])
        else:
          @pl.when(j + NB < CH_W)
          def _():
            wait_idx(j + NB, 0)
            gather(0, 1 - r, sem_g[1 - r])

        scatter(b, r)

        @pl.when(j + NB + b < CH_W)
        def _():
          fire_idx(j + NB + b, b)
      return carry

    lax.fori_loop(0, CH_W // NB, body, 0)

    plsc.subcore_barrier()
    _zero_or_dump(sid, None, acc, out_ref=out_hbm, cid=cid)

  return k


def _bn(h, gamma, beta, eps=1e-5):
  m = jnp.mean(h, axis=0, keepdims=True)
  v = jnp.mean((h - m) ** 2, axis=0, keepdims=True)
  return gamma * (h - m) * jax.lax.rsqrt(v + eps) + beta


def _k0_body(x_ref, wl_ref, wr_ref, b_ref, y_ref, z_ref):
  x = x_ref[...]
  n = x.shape[0]
  y = jnp.dot(x, wl_ref[...], preferred_element_type=jnp.float32)
  pad1 = jnp.ones((n, 1), jnp.float32)
  pad0 = jnp.zeros((n, 7), jnp.float32)
  y_ref[...] = jnp.concatenate([y, pad1, pad0], axis=1)
  z_ref[...] = jnp.dot(x, wr_ref[...], preferred_element_type=jnp.float32) + b_ref[...]


def _k0b_body(s_ref, z_ref, g_ref, be_ref, h_ref, inv_ref):
  s = s_ref[0] + s_ref[1]
  cnt = s[:, 64:65]
  inv = 1.0 / jnp.maximum(cnt, 1.0)
  inv_ref[...] = jnp.broadcast_to(inv, inv_ref.shape)
  h = s[:, :64] * inv + z_ref[...]
  h_ref[...] = jnp.maximum(_bn(h, g_ref[...], be_ref[...]), 0.0)


def _mid_body(h_ref, s_ref, inv_ref, wl_ref, wr_ref, b_ref, g_ref, be_ref, o_ref):
  inv = inv_ref[:, 0:1]
  mean = (s_ref[0] + s_ref[1]) * inv
  h = (jnp.dot(mean, wl_ref[...], preferred_element_type=jnp.float32)
       + jnp.dot(h_ref[...], wr_ref[...], preferred_element_type=jnp.float32)
       + b_ref[...])
  o_ref[...] = jnp.maximum(_bn(h, g_ref[...], be_ref[...]), 0.0)


def _wide_body(h_ref, sa_ref, sb_ref, inv_ref, wla_ref, wlb_ref, wr_ref,
               b_ref, g_ref, be_ref, o_ref):
  inv = inv_ref[:, 0:1]
  mean_a = (sa_ref[0] + sa_ref[1]) * inv
  mean_b = (sb_ref[0] + sb_ref[1]) * inv
  h = (jnp.dot(mean_a, wla_ref[...], preferred_element_type=jnp.float32)
       + jnp.dot(mean_b, wlb_ref[...], preferred_element_type=jnp.float32)
       + jnp.dot(h_ref[...], wr_ref[...], preferred_element_type=jnp.float32)
       + b_ref[...])
  o_ref[...] = jnp.maximum(_bn(h, g_ref[...], be_ref[...]), 0.0)


def _last_body(h_ref, sa_ref, sb_ref, inv_ref, wla_ref, wlb_ref, wr_ref,
               b_ref, o_ref):
  inv = inv_ref[:, 0:1]
  mean_a = (sa_ref[0] + sa_ref[1]) * inv
  mean_b = (sb_ref[0] + sb_ref[1]) * inv
  o_ref[...] = (jnp.dot(mean_a, wla_ref[...], preferred_element_type=jnp.float32)
                + jnp.dot(mean_b, wlb_ref[...], preferred_element_type=jnp.float32)
                + jnp.dot(h_ref[...], wr_ref[...], preferred_element_type=jnp.float32)
                + b_ref[...])


def kernel(x, edge_index, params):
  n = x.shape[0]
  f32 = jnp.float32
  pad = E_PAD - N_EDGES
  pad_ar = jnp.arange(pad, dtype=jnp.int32)
  src = jnp.concatenate([edge_index[0], pad_ar % N_NODES])
  dst = jnp.concatenate([edge_index[1], SINK + pad_ar % N_SINK])

  zeros72 = jnp.zeros((ROWS_Z, 72), f32)
  zeros64 = jnp.zeros((ROWS_Z, 64), f32)
  zeros128 = jnp.zeros((ROWS_Z, W), f32)
  scat72 = _make_feat_scatter(72, tc_tiling=False)
  scat64 = _make_feat_scatter(64, tc_tiling=False)
  scat128 = _make_feat_scatter(W, tc_tiling=True)
  sds = jax.ShapeDtypeStruct

  y0, z0 = pl.pallas_call(
      _k0_body,
      out_shape=(sds((n, 72), f32), sds((n, 64), f32)),
  )(x, params["Wl0"], params["Wr0"], params["b0"].reshape(1, 64))
  s0 = scat72(y0, src, dst, zeros72)
  h1, inv = pl.pallas_call(
      _k0b_body,
      out_shape=(sds((n, 64), f32), sds((n, 16), f32)),
  )(s0, z0, params["gamma0"].reshape(1, 64), params["beta0"].reshape(1, 64))

  s1 = scat64(h1, src, dst, zeros64)
  h2 = pl.pallas_call(
      _mid_body,
      out_shape=sds((n, 128), f32),
  )(h1, s1, inv, params["Wl1"], params["Wr1"], params["b1"].reshape(1, 128),
    params["gamma1"].reshape(1, 128), params["beta1"].reshape(1, 128))

  s2 = scat128(h2, src, dst, zeros128)
  h3 = pl.pallas_call(
      _mid_body,
      out_shape=sds((n, 256), f32),
  )(h2, s2, inv, params["Wl2"], params["Wr2"], params["b2"].reshape(1, 256),
    params["gamma2"].reshape(1, 256), params["beta2"].reshape(1, 256))

  s3a = scat128(h3[:, :128], src, dst, zeros128)
  s3b = scat128(h3[:, 128:], src, dst, zeros128)
  h4 = pl.pallas_call(
      _wide_body,
      out_shape=sds((n, 256), f32),
  )(h3, s3a, s3b, inv, params["Wl3"][:128], params["Wl3"][128:],
    params["Wr3"], params["b3"].reshape(1, 256),
    params["gamma3"].reshape(1, 256), params["beta3"].reshape(1, 256))

  s4a = scat128(h4[:, :128], src, dst, zeros128)
  s4b = scat128(h4[:, 128:], src, dst, zeros128)
  mb = 2000
  out = pl.pallas_call(
      _last_body,
      grid=(n // mb,),
      in_specs=[
          pl.BlockSpec((mb, 256), lambda i: (i, 0)),
          pl.BlockSpec((2, mb, 128), lambda i: (0, i, 0)),
          pl.BlockSpec((2, mb, 128), lambda i: (0, i, 0)),
          pl.BlockSpec((mb, 16), lambda i: (i, 0)),
          pl.BlockSpec((128, 576), lambda i: (0, 0)),
          pl.BlockSpec((128, 576), lambda i: (0, 0)),
          pl.BlockSpec((256, 576), lambda i: (0, 0)),
          pl.BlockSpec((1, 576), lambda i: (0, 0)),
      ],
      out_specs=pl.BlockSpec((mb, 576), lambda i: (i, 0)),
      out_shape=sds((n, 576), f32),
  )(h4, s4a, s4b, inv, params["Wl4"][:128], params["Wl4"][128:],
    params["Wr4"], params["b4"].reshape(1, 576))
  return out
